# Initial kernel scaffold; baseline (speedup 1.0000x reference)
import jax
import jax.numpy as jnp
from jax.experimental import pallas as pl


def kernel(x, W_router, W_up, b_up, W_down, b_down):
    raise NotImplementedError("write your pallas kernel here")


# speedup vs baseline: 1.2365x; 1.2365x over previous
import functools

import jax
import jax.numpy as jnp
from jax.experimental import pallas as pl
from jax.experimental.pallas import tpu as pltpu

_NUM_EXPERTS = 8
_TOP_K = 2
_CAPACITY_FACTOR = 1.0


def _ffn_body(buf_ref, wu_ref, bu_ref, wd_ref, bd_ref, o_ref):
    j = pl.program_id(1)
    xb = buf_ref[...].astype(jnp.bfloat16)
    up = jnp.dot(xb, wu_ref[0], preferred_element_type=jnp.float32)
    up = up + bu_ref[0]
    h = jax.nn.gelu(up)
    yp = jnp.dot(h.astype(jnp.bfloat16), wd_ref[0],
                 preferred_element_type=jnp.float32)

    @pl.when(j == 0)
    def _():
        o_ref[...] = yp + bd_ref[0]


    @pl.when(j != 0)
    def _():
        o_ref[...] += yp


def _expert_ffn(buf, wu, b_up, wd, b_down, ht):
    ec, d = buf.shape
    e, _, h = wu.shape
    c = ec // e
    grid = (e, h // ht)
    return pl.pallas_call(
        _ffn_body,
        grid=grid,
        in_specs=[
            pl.BlockSpec((c, d), lambda i, j: (i, 0)),
            pl.BlockSpec((1, d, ht), lambda i, j: (i, 0, j)),
            pl.BlockSpec((1, 1, ht), lambda i, j: (i, 0, j)),
            pl.BlockSpec((1, ht, d), lambda i, j: (i, j, 0)),
            pl.BlockSpec((1, 1, d), lambda i, j: (i, 0, 0)),
        ],
        out_specs=pl.BlockSpec((c, d), lambda i, j: (i, 0)),
        out_shape=jax.ShapeDtypeStruct((ec, d), jnp.float32),
        compiler_params=pltpu.CompilerParams(
            dimension_semantics=("arbitrary", "arbitrary"),
        ),
    )(buf, wu, b_up[:, None, :], wd, b_down[:, None, :])


def kernel(x, W_router, W_up, b_up, W_down, b_down):
    B, S, D = x.shape
    T = B * S
    E = _NUM_EXPERTS
    K = _TOP_K
    H = W_up.shape[2]
    C = max(int(_CAPACITY_FACTOR * T / E), K)

    xr = x.reshape(T, D)
    router_logits = xr @ W_router
    router_z_loss = jnp.mean(jnp.square(
        jax.nn.logsumexp(router_logits, axis=-1, keepdims=True)))
    router_probs = jax.nn.softmax(router_logits, axis=-1)
    top_k_probs, top_k_indices = jax.lax.top_k(router_probs, K)
    top_k_probs = top_k_probs / jnp.sum(top_k_probs, axis=-1, keepdims=True)
    sorted_idx = jnp.argsort(-1.0 * top_k_probs[:, 0])
    e_flat = top_k_indices[sorted_idx].reshape(-1)
    p_flat = top_k_probs[sorted_idx].reshape(-1)
    tok_flat = jnp.repeat(sorted_idx, K)
    one_hot = jax.nn.one_hot(e_flat, E, dtype=jnp.int32)
    pos = jnp.take_along_axis(jnp.cumsum(one_hot, axis=0),
                              e_flat[:, None], axis=1)[:, 0] - 1
    keep = pos < C
    slot = jnp.where(keep, pos, C)

    counts = jnp.sum(one_hot * keep[:, None].astype(jnp.int32), axis=0)
    frac = counts.astype(jnp.float32) / float(T * K)
    balance_loss = jnp.mean(jnp.square(frac - 1.0 / E))

    slot_id = jnp.where(keep, e_flat * C + slot, E * C)
    src = jnp.zeros(E * C + 1, jnp.int32).at[slot_id].set(tok_flat)[:E * C]
    gate = jnp.where(keep, p_flat, 0.0)
    a_tok = jnp.zeros((T, K), jnp.int32).at[sorted_idx].set(
        jnp.where(keep, slot_id, 0).reshape(T, K))
    g_tok = jnp.zeros((T, K), xr.dtype).at[sorted_idx].set(
        gate.reshape(T, K))

    buf = jnp.take(xr, src, axis=0)

    wu = W_up.astype(jnp.bfloat16)
    wd = W_down.astype(jnp.bfloat16)
    y2d = _expert_ffn(buf, wu, b_up, wd, b_down, ht=1024)

    yk = jnp.take(y2d, a_tok.reshape(-1), axis=0).reshape(T, K, D)
    out = jnp.einsum('tk,tkd->td', g_tok, yk)
    return out.reshape(B, S, D), router_z_loss, balance_loss


# speedup vs baseline: 1.5545x; 1.2571x over previous
import functools

import jax
import jax.numpy as jnp
from jax.experimental import pallas as pl
from jax.experimental.pallas import tpu as pltpu

_NUM_EXPERTS = 8
_TOP_K = 2
_CAPACITY_FACTOR = 1.0


def _ffn_body(buf_ref, wu_ref, bu_ref, wd_ref, bd_ref, o_ref):
    j = pl.program_id(1)
    xb = buf_ref[...].astype(jnp.bfloat16)
    up = jnp.dot(xb, wu_ref[0].astype(jnp.bfloat16),
                 preferred_element_type=jnp.float32)
    up = up + bu_ref[0]
    h = jax.nn.gelu(up)
    yp = jnp.dot(h.astype(jnp.bfloat16), wd_ref[0].astype(jnp.bfloat16),
                 preferred_element_type=jnp.float32)

    @pl.when(j == 0)
    def _():
        o_ref[...] = yp + bd_ref[0]


    @pl.when(j != 0)
    def _():
        o_ref[...] += yp


def _expert_ffn(buf, wu, b_up, wd, b_down, ht):
    ec, d = buf.shape
    e, _, h = wu.shape
    c = ec // e
    grid = (e, h // ht)
    return pl.pallas_call(
        _ffn_body,
        grid=grid,
        in_specs=[
            pl.BlockSpec((c, d), lambda i, j: (i, 0)),
            pl.BlockSpec((1, d, ht), lambda i, j: (i, 0, j)),
            pl.BlockSpec((1, 1, ht), lambda i, j: (i, 0, j)),
            pl.BlockSpec((1, ht, d), lambda i, j: (i, j, 0)),
            pl.BlockSpec((1, 1, d), lambda i, j: (i, 0, 0)),
        ],
        out_specs=pl.BlockSpec((c, d), lambda i, j: (i, 0)),
        out_shape=jax.ShapeDtypeStruct((ec, d), jnp.float32),
        compiler_params=pltpu.CompilerParams(
            dimension_semantics=("arbitrary", "arbitrary"),
        ),
    )(buf, wu, b_up[:, None, :], wd, b_down[:, None, :])


def kernel(x, W_router, W_up, b_up, W_down, b_down):
    B, S, D = x.shape
    T = B * S
    E = _NUM_EXPERTS
    K = _TOP_K
    H = W_up.shape[2]
    C = max(int(_CAPACITY_FACTOR * T / E), K)

    xr = x.reshape(T, D)
    router_logits = xr @ W_router
    router_z_loss = jnp.mean(jnp.square(
        jax.nn.logsumexp(router_logits, axis=-1, keepdims=True)))
    router_probs = jax.nn.softmax(router_logits, axis=-1)
    top_k_probs, top_k_indices = jax.lax.top_k(router_probs, K)
    top_k_probs = top_k_probs / jnp.sum(top_k_probs, axis=-1, keepdims=True)
    sorted_idx = jnp.argsort(-1.0 * top_k_probs[:, 0])
    e_flat = top_k_indices[sorted_idx].reshape(-1)
    p_flat = top_k_probs[sorted_idx].reshape(-1)
    tok_flat = jnp.repeat(sorted_idx, K)
    one_hot = jax.nn.one_hot(e_flat, E, dtype=jnp.int32)
    pos = jnp.take_along_axis(jnp.cumsum(one_hot, axis=0),
                              e_flat[:, None], axis=1)[:, 0] - 1
    keep = pos < C
    slot = jnp.where(keep, pos, C)

    counts = jnp.sum(one_hot * keep[:, None].astype(jnp.int32), axis=0)
    frac = counts.astype(jnp.float32) / float(T * K)
    balance_loss = jnp.mean(jnp.square(frac - 1.0 / E))

    slot_id = jnp.where(keep, e_flat * C + slot, E * C)
    src = jnp.zeros(E * C + 1, jnp.int32).at[slot_id].set(tok_flat)[:E * C]
    gate = jnp.where(keep, p_flat, 0.0)
    a_tok = jnp.zeros((T, K), jnp.int32).at[sorted_idx].set(
        jnp.where(keep, slot_id, 0).reshape(T, K))
    g_tok = jnp.zeros((T, K), xr.dtype).at[sorted_idx].set(
        gate.reshape(T, K))

    buf = jnp.take(xr, src, axis=0)

    y2d = _expert_ffn(buf, W_up, b_up, W_down, b_down, ht=1024)

    yk = jnp.take(y2d, a_tok.reshape(-1), axis=0).reshape(T, K, D)
    out = jnp.einsum('tk,tkd->td', g_tok, yk)
    return out.reshape(B, S, D), router_z_loss, balance_loss
